# baseline (speedup 1.0000x reference)
import jax
import jax.numpy as jnp
from jax.experimental import pallas as pl

HOP = 5


def _lrelu(v):
    return jnp.where(v > 0, v, 0.2 * v)


def _proj_body(h_ref, w_ref, b_ref, o_ref):
    o_ref[...] = h_ref[...] @ w_ref[...] + b_ref[...]


def kernel(x_author, x_paper, edge_index_ap, edge_index_pa, W1_author, b1_author, W1_paper, b1_paper, attn1, attn2, W_out, b_out):
    xa = jax.nn.relu(x_author @ W1_author + b1_author)
    xp = jax.nn.relu(x_paper @ W1_paper + b1_paper)
    x_dict = {"author": xa, "paper": xp}
    h_dict = {"author": xa, "paper": xp}
    edges = {("author", "paper"): edge_index_ap, ("paper", "author"): edge_index_pa}
    edge_types = [("author", "paper"), ("paper", "author")]
    for i in range(HOP):
        aggr = {}
        for j, (st, tt) in enumerate(edge_types):
            ei = edges[(st, tt)]
            s, t = ei[0], ei[1]
            x = x_dict[st]
            h = h_dict[tt]
            N = x.shape[0]
            a1 = attn1[i, j]
            a2 = attn2[i, j]
            x1 = x @ a1
            h1 = h @ a2
            w2 = x1 + x @ a2
            w1 = jnp.exp(_lrelu(x1[s] + h1[t]))
            w2 = jnp.exp(_lrelu(w2))
            div = jax.ops.segment_sum(w1, s, num_segments=N) + w2
            hn = jax.ops.segment_sum(w1[:, None] * h[t], s, num_segments=N) + w2[:, None] * x
            hn = hn / div[:, None]
            aggr[st] = hn
        for nt in aggr:
            h_dict[nt] = jax.nn.elu(aggr[nt])
    h = h_dict["author"]
    N, H = h.shape
    OUT = W_out.shape[1]
    BR = 400
    return pl.pallas_call(
        _proj_body,
        grid=(N // BR,),
        in_specs=[
            pl.BlockSpec((BR, H), lambda i: (i, 0)),
            pl.BlockSpec((H, OUT), lambda i: (0, 0)),
            pl.BlockSpec((OUT,), lambda i: (0,)),
        ],
        out_specs=pl.BlockSpec((BR, OUT), lambda i: (i, 0)),
        out_shape=jax.ShapeDtypeStruct((N, OUT), jnp.float32),
    )(h, W_out, b_out)


# speedup vs baseline: 3.5891x; 3.5891x over previous
import functools

import jax
import jax.numpy as jnp
from jax import lax
from jax.experimental import pallas as pl
from jax.experimental.pallas import tpu as pltpu
from jax.experimental.pallas import tpu_sc as plsc

HOP = 5
N = 10000
NPAD = 10240
H = 128
HH = 64
E = 320000
EPAD = 327680
CH = 512
NCH = 40
TILE_ROWS = 160
SLAB = NPAD // 16
DUMMY = 10016


def _lrelu(v):
    return jnp.where(v > 0, v, 0.2 * v)


def _agg_kernel(hap0, hap1, hpa0, hpa1, s_ap, t_ap, s_pa, t_pa, scal_ap, scal_pa,
                accap0, accap1, seg_ap, accpa0, accpa1, seg_pa,
                x1buf, h1buf, sidx, tidx, rows, wbuf, zseg, accsh, segsh, sem):
    sid = lax.axis_index("s")
    cid = lax.axis_index("c")
    zero16 = jnp.zeros((16,), jnp.float32)

    def _zero_rows():
        def _z(i, c):
            for q in range(HH // 16):
                rows[i, pl.ds(q * 16, 16)] = zero16
            return c
        lax.fori_loop(0, CH, _z, 0)

    def _run_type(h0, h1, s2d, t2d, scal, acc0_out, acc1_out, seg_out):
        pltpu.sync_copy(scal.at[0], x1buf)
        pltpu.sync_copy(scal.at[1], h1buf)

        def _zseg(i, c):
            zseg[pl.ds(i * 16, 16)] = zero16
            return c
        lax.fori_loop(0, SLAB // 16, _zseg, 0)

        tile_row0 = sid * TILE_ROWS

        for p in range(2):
            hs = h0 if p == 0 else h1
            acc_out = acc0_out if p == 0 else acc1_out
            _zero_rows()
            pltpu.sync_copy(rows, accsh.at[pl.ds(sid * SLAB, CH)])
            pltpu.sync_copy(rows.at[pl.ds(0, SLAB - CH)],
                            accsh.at[pl.ds(sid * SLAB + CH, SLAB - CH)])
            if p == 0:
                pltpu.sync_copy(zseg, segsh.at[pl.ds(sid * SLAB, SLAB)])
            plsc.subcore_barrier()

            def _chunk(c, carry):
                base = tile_row0 + c * 4
                pltpu.sync_copy(s2d.at[pl.ds(base, 4)], sidx)
                pltpu.sync_copy(t2d.at[pl.ds(base, 4)], tidx)
                cps = [pltpu.async_copy(hs.at[tidx.at[j]],
                                        rows.at[pl.ds(j * 128, 128)], sem)
                       for j in range(4)]
                for cp in cps:
                    cp.wait()
                if p == 0:
                    def _wg(g, cc):
                        r = g // 8
                        col = (g % 8) * 16
                        sv = sidx[r, pl.ds(col, 16)]
                        tv = tidx[r, pl.ds(col, 16)]
                        xv = plsc.load_gather(x1buf, [sv])
                        hv = plsc.load_gather(h1buf, [tv])
                        wbuf[c * 4 + r, pl.ds(col, 16)] = (
                            jnp.exp(_lrelu(xv + hv)))
                        return cc
                    lax.fori_loop(0, 32, _wg, 0)
                def _scale(g, cc):
                    wv = wbuf[c * 4 + g // 8, pl.ds((g % 8) * 16, 16)]
                    for k in range(16):
                        e = g * 16 + k
                        w = wv[k]
                        for q in range(HH // 16):
                            rows[e, pl.ds(q * 16, 16)] = (
                                rows[e, pl.ds(q * 16, 16)] * w)
                    return cc
                lax.fori_loop(0, 32, _scale, 0)
                for j in range(4):
                    pltpu.sync_copy(rows.at[pl.ds(j * 128, 128)],
                                    accsh.at[sidx.at[j]], add=True)
                    if p == 0:
                        pltpu.sync_copy(wbuf.at[c * 4 + j],
                                        segsh.at[sidx.at[j]], add=True)
                return carry

            lax.fori_loop(0, NCH, _chunk, 0)
            plsc.subcore_barrier()
            pltpu.sync_copy(accsh.at[pl.ds(sid * SLAB, SLAB)],
                            acc_out.at[pl.ds(sid * SLAB, SLAB)])
            if p == 0:
                pltpu.sync_copy(segsh.at[pl.ds(sid * SLAB, SLAB)],
                                seg_out.at[pl.ds(sid * SLAB, SLAB)])

    @pl.when(cid == 0)
    def _():
        _run_type(hap0, hap1, s_ap, t_ap, scal_ap, accap0, accap1, seg_ap)

    @pl.when(cid == 1)
    def _():
        _run_type(hpa0, hpa1, s_pa, t_pa, scal_pa, accpa0, accpa1, seg_pa)


_agg = functools.partial(
    pl.kernel,
    mesh=plsc.VectorSubcoreMesh(core_axis_name="c", subcore_axis_name="s"),
    compiler_params=pltpu.CompilerParams(
        needs_layout_passes=False, use_tc_tiling_on_sc=False),
    out_type=[
        jax.ShapeDtypeStruct((NPAD, HH), jnp.float32),
        jax.ShapeDtypeStruct((NPAD, HH), jnp.float32),
        jax.ShapeDtypeStruct((NPAD,), jnp.float32),
        jax.ShapeDtypeStruct((NPAD, HH), jnp.float32),
        jax.ShapeDtypeStruct((NPAD, HH), jnp.float32),
        jax.ShapeDtypeStruct((NPAD,), jnp.float32),
    ],
    scratch_types=[
        pltpu.VMEM((NPAD,), jnp.float32),
        pltpu.VMEM((NPAD,), jnp.float32),
        pltpu.VMEM((4, 128), jnp.int32),
        pltpu.VMEM((4, 128), jnp.int32),
        pltpu.VMEM((CH, HH), jnp.float32),
        pltpu.VMEM((TILE_ROWS, 128), jnp.float32),
        pltpu.VMEM((SLAB,), jnp.float32),
        pltpu.VMEM_SHARED((NPAD, HH), jnp.float32),
        pltpu.VMEM_SHARED((NPAD,), jnp.float32),
        pltpu.SemaphoreType.DMA,
    ],
)(_agg_kernel)


def _proj_body(h_ref, w_ref, b_ref, o_ref):
    o_ref[...] = h_ref[...] @ w_ref[...] + b_ref[...]


def _pad_edges(ei):
    s = jnp.concatenate([ei[0], jnp.full((EPAD - E,), DUMMY, jnp.int32)])
    t = jnp.concatenate([ei[1], jnp.zeros((EPAD - E,), jnp.int32)])
    return s.reshape(EPAD // 128, 128), t.reshape(EPAD // 128, 128)


def _pad_vec(v):
    return jnp.concatenate([v, jnp.zeros((NPAD - N,), jnp.float32)])


def kernel(x_author, x_paper, edge_index_ap, edge_index_pa, W1_author, b1_author, W1_paper, b1_paper, attn1, attn2, W_out, b_out):
    xa = jax.nn.relu(x_author @ W1_author + b1_author)
    xp = jax.nn.relu(x_paper @ W1_paper + b1_paper)
    s_ap, t_ap = _pad_edges(edge_index_ap)
    s_pa, t_pa = _pad_edges(edge_index_pa)

    A1_ap = attn1[:, 0, :].T
    A2_ap = attn2[:, 0, :].T
    A1_pa = attn1[:, 1, :].T
    A2_pa = attn2[:, 1, :].T
    x1_ap_all = xa @ A1_ap
    x1_pa_all = xp @ A1_pa
    w2_ap_all = jnp.exp(_lrelu(x1_ap_all + xa @ A2_ap))
    w2_pa_all = jnp.exp(_lrelu(x1_pa_all + xp @ A2_pa))

    ha, hp = xa, xp
    for i in range(HOP):
        h1_ap = _pad_vec(hp @ attn2[i, 0])
        h1_pa = _pad_vec(ha @ attn2[i, 1])
        scal_ap = jnp.stack([_pad_vec(x1_ap_all[:, i]), h1_ap])
        scal_pa = jnp.stack([_pad_vec(x1_pa_all[:, i]), h1_pa])
        acc_ap0, acc_ap1, seg_ap, acc_pa0, acc_pa1, seg_pa = _agg(
            hp[:, :HH], hp[:, HH:], ha[:, :HH], ha[:, HH:],
            s_ap, t_ap, s_pa, t_pa, scal_ap, scal_pa)
        acc_a = jnp.concatenate([acc_ap0[:N], acc_ap1[:N]], axis=1)
        acc_p = jnp.concatenate([acc_pa0[:N], acc_pa1[:N]], axis=1)
        w2a = w2_ap_all[:, i]
        w2p = w2_pa_all[:, i]
        hn_a = (acc_a + w2a[:, None] * xa) / (seg_ap[:N] + w2a)[:, None]
        hn_p = (acc_p + w2p[:, None] * xp) / (seg_pa[:N] + w2p)[:, None]
        ha = jax.nn.elu(hn_a)
        hp = jax.nn.elu(hn_p)

    BR = 400
    OUT = W_out.shape[1]
    return pl.pallas_call(
        _proj_body,
        grid=(N // BR,),
        in_specs=[
            pl.BlockSpec((BR, H), lambda i: (i, 0)),
            pl.BlockSpec((H, OUT), lambda i: (0, 0)),
            pl.BlockSpec((OUT,), lambda i: (0,)),
        ],
        out_specs=pl.BlockSpec((BR, OUT), lambda i: (i, 0)),
        out_shape=jax.ShapeDtypeStruct((N, OUT), jnp.float32),
    )(ha, W_out, b_out)


# speedup vs baseline: 4.9544x; 1.3804x over previous
import functools

import jax
import jax.numpy as jnp
from jax import lax
from jax.experimental import pallas as pl
from jax.experimental.pallas import tpu as pltpu
from jax.experimental.pallas import tpu_sc as plsc

HOP = 5
N = 10000
NPAD = 10240
H = 128
HH = 64
E = 320000
CH = 128
NCH = 162
NPAIR = NCH // 3
EPAD = 16 * NCH * CH
SLAB = NPAD // 16
DUMMY = 10016


def _lrelu(v):
    return jnp.where(v > 0, v, 0.2 * v)


def _agg_kernel(hap0, hap1, hpa0, hpa1, s_ap, t_ap, s_pa, t_pa, scal_ap, scal_pa,
                accap0, accap1, seg_ap, accpa0, accpa1, seg_pa,
                x1buf, h1buf, sidx, tidx, rows0, rows1, rows2, wbuf,
                accsh, segsh, gsem0, gsem1, gsem2, ssem0, ssem1, ssem2):
    sid = lax.axis_index("s")
    cid = lax.axis_index("c")
    zero16 = jnp.zeros((16,), jnp.float32)
    rows_bufs = (rows0, rows1, rows2)
    gsems = (gsem0, gsem1, gsem2)
    ssems = (ssem0, ssem1, ssem2)

    def _run_type(h0, h1, s2d, t2d, scal, acc0_out, acc1_out, seg_out):
        pltpu.sync_copy(scal.at[0], x1buf)
        pltpu.sync_copy(scal.at[1], h1buf)
        tile_row0 = sid * NCH
        pltpu.sync_copy(s2d.at[pl.ds(tile_row0, NCH)], sidx)
        pltpu.sync_copy(t2d.at[pl.ds(tile_row0, NCH)], tidx)

        for p in range(2):
            hs = h0 if p == 0 else h1
            acc_out = acc0_out if p == 0 else acc1_out

            def _fire_gather(c, b):
                return pltpu.async_copy(hs.at[tidx.at[c]], rows_bufs[b],
                                        gsems[b])

            def _wait_gather(c, b):
                pltpu.make_async_copy(hs.at[tidx.at[c]], rows_bufs[b],
                                      gsems[b]).wait()

            def _fire_scatter(c, b):
                pltpu.async_copy(rows_bufs[b], accsh.at[sidx.at[c]],
                                 ssems[b], add=True)
                if p == 0:
                    pltpu.async_copy(wbuf.at[b], segsh.at[sidx.at[c]],
                                     ssems[b], add=True)

            def _wait_scatter(c, b):
                pltpu.make_async_copy(rows_bufs[b], accsh.at[sidx.at[c]],
                                      ssems[b]).wait()
                if p == 0:
                    pltpu.make_async_copy(wbuf.at[b], segsh.at[sidx.at[c]],
                                          ssems[b]).wait()

            def _zrow(i, cc):
                for q in range(HH // 16):
                    rows0[i, pl.ds(q * 16, 16)] = zero16
                return cc
            lax.fori_loop(0, CH, _zrow, 0)
            for r in range(SLAB // CH):
                pltpu.sync_copy(rows0,
                                accsh.at[pl.ds(sid * SLAB + r * CH, CH)])
            if p == 0:
                for q in range(8):
                    wbuf[0, pl.ds(q * 16, 16)] = zero16
                for r in range(SLAB // 128):
                    pltpu.sync_copy(wbuf.at[0],
                                    segsh.at[pl.ds(sid * SLAB + r * 128, 128)])
            plsc.subcore_barrier()

            _fire_gather(0, 0)

            def _pair(g, carry):
                for db in range(3):
                    c = g * 3 + db
                    b = db
                    nb = (db + 1) % 3
                    if db < 2:
                        @pl.when(c >= 2)
                        def _():
                            _wait_scatter(c - 2, nb)
                        _fire_gather(c + 1, nb)
                    else:
                        @pl.when(g < NPAIR - 1)
                        def _():
                            _wait_scatter(c - 2, nb)
                            _fire_gather(c + 1, nb)
                    _wait_gather(c, b)
                    rows = rows_bufs[b]
                    def _wg(gg, cc):
                        col = gg * 16
                        sv = sidx[c, pl.ds(col, 16)]
                        tv = tidx[c, pl.ds(col, 16)]
                        xv = plsc.load_gather(x1buf, [sv])
                        hv = plsc.load_gather(h1buf, [tv])
                        wbuf[b, pl.ds(col, 16)] = (
                            jnp.exp(_lrelu(xv + hv)))
                        return cc
                    lax.fori_loop(0, 8, _wg, 0)
                    def _scale(gg, cc):
                        wv = wbuf[b, pl.ds(gg * 16, 16)]
                        for k in range(16):
                            e = gg * 16 + k
                            w = wv[k]
                            for q in range(HH // 16):
                                rows[e, pl.ds(q * 16, 16)] = (
                                    rows[e, pl.ds(q * 16, 16)] * w)
                        return cc
                    lax.fori_loop(0, 8, _scale, 0)
                    _fire_scatter(c, b)
                return carry

            lax.fori_loop(0, NPAIR, _pair, 0)
            for c, b in ((NCH - 3, 0), (NCH - 2, 1), (NCH - 1, 2)):
                _wait_scatter(c, b)
            plsc.subcore_barrier()
            pltpu.sync_copy(accsh.at[pl.ds(sid * SLAB, SLAB)],
                            acc_out.at[pl.ds(sid * SLAB, SLAB)])
            if p == 0:
                pltpu.sync_copy(segsh.at[pl.ds(sid * SLAB, SLAB)],
                                seg_out.at[pl.ds(sid * SLAB, SLAB)])

    @pl.when(cid == 0)
    def _():
        _run_type(hap0, hap1, s_ap, t_ap, scal_ap, accap0, accap1, seg_ap)

    @pl.when(cid == 1)
    def _():
        _run_type(hpa0, hpa1, s_pa, t_pa, scal_pa, accpa0, accpa1, seg_pa)


_agg = functools.partial(
    pl.kernel,
    mesh=plsc.VectorSubcoreMesh(core_axis_name="c", subcore_axis_name="s"),
    compiler_params=pltpu.CompilerParams(
        needs_layout_passes=False, use_tc_tiling_on_sc=False),
    out_type=[
        jax.ShapeDtypeStruct((NPAD, HH), jnp.float32),
        jax.ShapeDtypeStruct((NPAD, HH), jnp.float32),
        jax.ShapeDtypeStruct((NPAD,), jnp.float32),
        jax.ShapeDtypeStruct((NPAD, HH), jnp.float32),
        jax.ShapeDtypeStruct((NPAD, HH), jnp.float32),
        jax.ShapeDtypeStruct((NPAD,), jnp.float32),
    ],
    scratch_types=[
        pltpu.VMEM((NPAD,), jnp.float32),
        pltpu.VMEM((NPAD,), jnp.float32),
        pltpu.VMEM((NCH, 128), jnp.int32),
        pltpu.VMEM((NCH, 128), jnp.int32),
        pltpu.VMEM((CH, HH), jnp.float32),
        pltpu.VMEM((CH, HH), jnp.float32),
        pltpu.VMEM((CH, HH), jnp.float32),
        pltpu.VMEM((3, 128), jnp.float32),
        pltpu.VMEM_SHARED((NPAD, HH), jnp.float32),
        pltpu.VMEM_SHARED((NPAD,), jnp.float32),
        pltpu.SemaphoreType.DMA,
        pltpu.SemaphoreType.DMA,
        pltpu.SemaphoreType.DMA,
        pltpu.SemaphoreType.DMA,
        pltpu.SemaphoreType.DMA,
        pltpu.SemaphoreType.DMA,
    ],
)(_agg_kernel)


def _proj_body(h_ref, w_ref, b_ref, o_ref):
    o_ref[...] = h_ref[...] @ w_ref[...] + b_ref[...]


def _pad_edges(ei):
    s = jnp.concatenate([ei[0], jnp.full((EPAD - E,), DUMMY, jnp.int32)])
    t = jnp.concatenate([ei[1], jnp.zeros((EPAD - E,), jnp.int32)])
    return s.reshape(EPAD // 128, 128), t.reshape(EPAD // 128, 128)


def _pad_vec(v):
    return jnp.concatenate([v, jnp.zeros((NPAD - N,), jnp.float32)])


def kernel(x_author, x_paper, edge_index_ap, edge_index_pa, W1_author, b1_author, W1_paper, b1_paper, attn1, attn2, W_out, b_out):
    xa = jax.nn.relu(x_author @ W1_author + b1_author)
    xp = jax.nn.relu(x_paper @ W1_paper + b1_paper)
    s_ap, t_ap = _pad_edges(edge_index_ap)
    s_pa, t_pa = _pad_edges(edge_index_pa)

    A1_ap = attn1[:, 0, :].T
    A2_ap = attn2[:, 0, :].T
    A1_pa = attn1[:, 1, :].T
    A2_pa = attn2[:, 1, :].T
    x1_ap_all = xa @ A1_ap
    x1_pa_all = xp @ A1_pa
    w2_ap_all = jnp.exp(_lrelu(x1_ap_all + xa @ A2_ap))
    w2_pa_all = jnp.exp(_lrelu(x1_pa_all + xp @ A2_pa))

    ha, hp = xa, xp
    for i in range(HOP):
        h1_ap = _pad_vec(hp @ attn2[i, 0])
        h1_pa = _pad_vec(ha @ attn2[i, 1])
        scal_ap = jnp.stack([_pad_vec(x1_ap_all[:, i]), h1_ap])
        scal_pa = jnp.stack([_pad_vec(x1_pa_all[:, i]), h1_pa])
        acc_ap0, acc_ap1, seg_ap, acc_pa0, acc_pa1, seg_pa = _agg(
            hp[:, :HH], hp[:, HH:], ha[:, :HH], ha[:, HH:],
            s_ap, t_ap, s_pa, t_pa, scal_ap, scal_pa)
        acc_a = jnp.concatenate([acc_ap0[:N], acc_ap1[:N]], axis=1)
        acc_p = jnp.concatenate([acc_pa0[:N], acc_pa1[:N]], axis=1)
        w2a = w2_ap_all[:, i]
        w2p = w2_pa_all[:, i]
        hn_a = (acc_a + w2a[:, None] * xa) / (seg_ap[:N] + w2a)[:, None]
        hn_p = (acc_p + w2p[:, None] * xp) / (seg_pa[:N] + w2p)[:, None]
        ha = jax.nn.elu(hn_a)
        hp = jax.nn.elu(hn_p)

    BR = 400
    OUT = W_out.shape[1]
    return pl.pallas_call(
        _proj_body,
        grid=(N // BR,),
        in_specs=[
            pl.BlockSpec((BR, H), lambda i: (i, 0)),
            pl.BlockSpec((H, OUT), lambda i: (0, 0)),
            pl.BlockSpec((OUT,), lambda i: (0,)),
        ],
        out_specs=pl.BlockSpec((BR, OUT), lambda i: (i, 0)),
        out_shape=jax.ShapeDtypeStruct((N, OUT), jnp.float32),
    )(ha, W_out, b_out)
